# baseline (speedup 1.0000x reference)
import functools

import jax
import jax.numpy as jnp
from jax.experimental import pallas as pl
from jax.experimental.pallas import tpu as pltpu

EPS = 1e-5


def _body(x_ref, adj_ref, w1_ref, b1_ref, w2_ref, b2_ref, bnp_ref,
          out_ref, y1_all, y2_all, adj_bf, acc, aff, *, nb, count):
    i = pl.program_id(0)
    f32 = jnp.float32

    def row_sums(y):
        ones = jnp.ones((y.shape[1], 8), f32)
        s = jnp.dot(y, ones, preferred_element_type=f32)
        q = jnp.dot(y * y, ones, preferred_element_type=f32)
        return s[:, 0:1], q[:, 0:1]

    def stats1_of(b_idx):
        z = y1_all[pl.ds(b_idx, 1)][0].astype(f32)
        y = jnp.maximum(z + b1_ref[...], 0.0)
        return row_sums(y)

    def stats2_of(b_idx):
        z = y2_all[pl.ds(b_idx, 1)][0].astype(f32)
        y = jnp.maximum(z + b2_ref[...], 0.0)
        return row_sums(y)

    def produce2(q, stashed):
        z = y1_all[pl.ds(q, 1)][0].astype(f32)
        h = (jnp.maximum(z + b1_ref[...], 0.0) * aff[:, 0:1] + aff[:, 1:2])
        s2v = jnp.dot(h, w2_ref[...], preferred_element_type=f32)
        if stashed:
            raw2 = jnp.dot(adj_bf[pl.ds(q - 1, 1)][0],
                           s2v.astype(jnp.bfloat16),
                           preferred_element_type=f32)
        else:
            raw2 = jnp.dot(adj_ref[0], s2v, preferred_element_type=f32)
        y2_all[pl.ds(q, 1)] = raw2[None].astype(jnp.bfloat16)

    def normalize(q):
        z = y2_all[pl.ds(q, 1)][0].astype(f32)
        out_ref[0] = (jnp.maximum(z + b2_ref[...], 0.0) * aff[:, 2:3]
                      + aff[:, 3:4])

    @pl.when(i < nb)
    def _phase0():
        ps, pq = stats1_of(jnp.maximum(i - 1, 0))
        acc[:, 0:1] = (jnp.where(i == 0, 0.0, acc[:, 0:1])
                       + jnp.where(i >= 1, ps, 0.0))
        acc[:, 1:2] = (jnp.where(i == 0, 0.0, acc[:, 1:2])
                       + jnp.where(i >= 1, pq, 0.0))
        s = jnp.dot(x_ref[0], w1_ref[...], preferred_element_type=f32)
        adj_b = adj_ref[0].astype(jnp.bfloat16)
        raw = jnp.dot(adj_b, s.astype(jnp.bfloat16),
                      preferred_element_type=f32)
        y1_all[pl.ds(i, 1)] = raw[None].astype(jnp.bfloat16)

        @pl.when((i >= 1) & (i <= nb - 2))
        def _stash():
            adj_bf[pl.ds(i - 1, 1)] = adj_b[None]

    @pl.when(i == nb)
    def _fin1():
        ps, pq = stats1_of(nb - 1)
        inv = 1.0 / count
        mean = (acc[:, 0:1] + ps) * inv
        var = (acc[:, 1:2] + pq) * inv - mean * mean
        a = bnp_ref[:, 0:1] * jax.lax.rsqrt(var + EPS)
        aff[:, 0:1] = a
        aff[:, 1:2] = bnp_ref[:, 1:2] - mean * a
        produce2(nb - 1, stashed=False)

    @pl.when(i == nb + 1)
    def _phase1_first():
        ps, pq = stats2_of(nb - 1)
        acc[:, 2:3] = ps
        acc[:, 3:4] = pq
        produce2(0, stashed=False)

    @pl.when((i > nb + 1) & (i < 2 * nb))
    def _phase1():
        j = i - nb
        ps, pq = stats2_of(j - 2)
        acc[:, 2:3] += ps
        acc[:, 3:4] += pq
        produce2(j - 1, stashed=True)

    @pl.when(i == 2 * nb)
    def _fin2():
        ps, pq = stats2_of(nb - 2)
        inv = 1.0 / count
        mean = (acc[:, 2:3] + ps) * inv
        var = (acc[:, 3:4] + pq) * inv - mean * mean
        a = bnp_ref[:, 2:3] * jax.lax.rsqrt(var + EPS)
        aff[:, 2:3] = a
        aff[:, 3:4] = bnp_ref[:, 3:4] - mean * a
        normalize(0)

    @pl.when(i > 2 * nb)
    def _norm():
        normalize(i - 2 * nb)


@jax.jit
def kernel(x, adj, W1, b1, W2, b2, gamma1, beta1, gamma2, beta2):
    B, N, C_in = x.shape
    C_hid = W1.shape[1]
    C_out = W2.shape[1]
    f32 = jnp.float32
    nb = B

    full = lambda shape: pl.BlockSpec(shape, lambda i: (0,) * len(shape))

    def adj_idx(i):
        return (jnp.where(i < nb, i, jnp.where(i == nb, nb - 1, 0)), 0, 0)

    bn_params = jnp.stack([gamma1, beta1, gamma2, beta2], axis=1)

    out = pl.pallas_call(
        functools.partial(_body, nb=nb, count=B * C_hid),
        grid=(3 * nb,),
        in_specs=[
            pl.BlockSpec((1, N, C_in), lambda i: (jnp.minimum(i, nb - 1), 0, 0)),
            pl.BlockSpec((1, N, N), adj_idx),
            full((C_in, C_hid)), full((1, C_hid)),
            full((C_hid, C_out)), full((1, C_out)),
            full((N, 4)),
        ],
        out_specs=pl.BlockSpec(
            (1, N, C_out), lambda i: (jnp.maximum(i - 2 * nb, 0), 0, 0)),
        out_shape=jax.ShapeDtypeStruct((B, N, C_out), f32),
        scratch_shapes=[
            pltpu.VMEM((B, N, C_hid), jnp.bfloat16),
            pltpu.VMEM((B, N, C_out), jnp.bfloat16),
            pltpu.VMEM((B - 2, N, N), jnp.bfloat16),
            pltpu.VMEM((N, 4), f32),
            pltpu.VMEM((N, 4), f32),
        ],
        compiler_params=pltpu.CompilerParams(
            vmem_limit_bytes=100 * 1024 * 1024),
    )(x, adj, W1, b1.reshape(1, C_hid), W2, b2.reshape(1, C_out), bn_params)

    return out


# speedup vs baseline: 1.2073x; 1.2073x over previous
import functools

import jax
import jax.numpy as jnp
from jax.experimental import pallas as pl
from jax.experimental.pallas import tpu as pltpu

EPS = 1e-5


def _body(x_ref, adj_ref, w1_ref, b1_ref, w2_ref, b2_ref, bnp_ref,
          out_ref, y1_all, y2_all, adj_bf, acc, aff, *, nb, count):
    i = pl.program_id(0)
    f32 = jnp.float32

    def stats1_of(b_idx):
        z = y1_all[pl.ds(b_idx, 1)][0].astype(f32)
        y = jnp.maximum(z + b1_ref[...], 0.0)
        return (jnp.sum(y, axis=1, keepdims=True),
                jnp.sum(y * y, axis=1, keepdims=True))

    def stats2_of(b_idx):
        z = y2_all[pl.ds(b_idx, 1)][0].astype(f32)
        y = jnp.maximum(z + b2_ref[...], 0.0)
        return (jnp.sum(y, axis=1, keepdims=True),
                jnp.sum(y * y, axis=1, keepdims=True))

    def produce2(q, stashed):
        z = y1_all[pl.ds(q, 1)][0].astype(f32)
        h = (jnp.maximum(z + b1_ref[...], 0.0) * aff[:, 0:1] + aff[:, 1:2])
        s2v = jnp.dot(h, w2_ref[...], preferred_element_type=f32)
        if stashed:
            raw2 = jnp.dot(adj_bf[pl.ds(q - 1, 1)][0],
                           s2v.astype(jnp.bfloat16),
                           preferred_element_type=f32)
        else:
            raw2 = jnp.dot(adj_ref[0], s2v, preferred_element_type=f32)
        y2_all[pl.ds(q, 1)] = raw2[None].astype(jnp.bfloat16)

    def normalize(q):
        z = y2_all[pl.ds(q, 1)][0].astype(f32)
        out_ref[0] = (jnp.maximum(z + b2_ref[...], 0.0) * aff[:, 2:3]
                      + aff[:, 3:4])

    @pl.when(i < nb)
    def _phase0():
        ps, pq = stats1_of(jnp.maximum(i - 1, 0))
        acc[:, 0:1] = (jnp.where(i == 0, 0.0, acc[:, 0:1])
                       + jnp.where(i >= 1, ps, 0.0))
        acc[:, 1:2] = (jnp.where(i == 0, 0.0, acc[:, 1:2])
                       + jnp.where(i >= 1, pq, 0.0))
        s = jnp.dot(x_ref[0], w1_ref[...], preferred_element_type=f32)
        adj_b = adj_ref[0].astype(jnp.bfloat16)
        raw = jnp.dot(adj_b, s.astype(jnp.bfloat16),
                      preferred_element_type=f32)
        y1_all[pl.ds(i, 1)] = raw[None].astype(jnp.bfloat16)

        @pl.when((i >= 1) & (i <= nb - 2))
        def _stash():
            adj_bf[pl.ds(i - 1, 1)] = adj_b[None]

    @pl.when(i == nb)
    def _fin1():
        ps, pq = stats1_of(nb - 1)
        inv = 1.0 / count
        mean = (acc[:, 0:1] + ps) * inv
        var = (acc[:, 1:2] + pq) * inv - mean * mean
        a = bnp_ref[:, 0:1] * jax.lax.rsqrt(var + EPS)
        aff[:, 0:1] = a
        aff[:, 1:2] = bnp_ref[:, 1:2] - mean * a
        produce2(nb - 1, stashed=False)

    @pl.when(i == nb + 1)
    def _phase1_first():
        ps, pq = stats2_of(nb - 1)
        acc[:, 2:3] = ps
        acc[:, 3:4] = pq
        produce2(0, stashed=False)

    @pl.when((i > nb + 1) & (i < 2 * nb))
    def _phase1():
        j = i - nb
        ps, pq = stats2_of(j - 2)
        acc[:, 2:3] += ps
        acc[:, 3:4] += pq
        produce2(j - 1, stashed=True)

    @pl.when(i == 2 * nb)
    def _fin2():
        ps, pq = stats2_of(nb - 2)
        inv = 1.0 / count
        mean = (acc[:, 2:3] + ps) * inv
        var = (acc[:, 3:4] + pq) * inv - mean * mean
        a = bnp_ref[:, 2:3] * jax.lax.rsqrt(var + EPS)
        aff[:, 2:3] = a
        aff[:, 3:4] = bnp_ref[:, 3:4] - mean * a
        normalize(0)

    @pl.when(i > 2 * nb)
    def _norm():
        normalize(i - 2 * nb)


@jax.jit
def kernel(x, adj, W1, b1, W2, b2, gamma1, beta1, gamma2, beta2):
    B, N, C_in = x.shape
    C_hid = W1.shape[1]
    C_out = W2.shape[1]
    f32 = jnp.float32
    nb = B

    full = lambda shape: pl.BlockSpec(shape, lambda i: (0,) * len(shape))

    def adj_idx(i):
        return (jnp.where(i < nb, i, jnp.where(i == nb, nb - 1, 0)), 0, 0)

    bn_params = jnp.stack([gamma1, beta1, gamma2, beta2], axis=1)

    out = pl.pallas_call(
        functools.partial(_body, nb=nb, count=B * C_hid),
        grid=(3 * nb,),
        in_specs=[
            pl.BlockSpec((1, N, C_in), lambda i: (jnp.minimum(i, nb - 1), 0, 0)),
            pl.BlockSpec((1, N, N), adj_idx),
            full((C_in, C_hid)), full((1, C_hid)),
            full((C_hid, C_out)), full((1, C_out)),
            full((N, 4)),
        ],
        out_specs=pl.BlockSpec(
            (1, N, C_out), lambda i: (jnp.maximum(i - 2 * nb, 0), 0, 0)),
        out_shape=jax.ShapeDtypeStruct((B, N, C_out), f32),
        scratch_shapes=[
            pltpu.VMEM((B, N, C_hid), jnp.bfloat16),
            pltpu.VMEM((B, N, C_out), jnp.bfloat16),
            pltpu.VMEM((B - 2, N, N), jnp.bfloat16),
            pltpu.VMEM((N, 4), f32),
            pltpu.VMEM((N, 4), f32),
        ],
        compiler_params=pltpu.CompilerParams(
            vmem_limit_bytes=100 * 1024 * 1024),
    )(x, adj, W1, b1.reshape(1, C_hid), W2, b2.reshape(1, C_out), bn_params)

    return out


# speedup vs baseline: 1.2172x; 1.0081x over previous
import functools

import jax
import jax.numpy as jnp
from jax.experimental import pallas as pl
from jax.experimental.pallas import tpu as pltpu

EPS = 1e-5


def _body(x_ref, adj_ref, w1_ref, b1_ref, w2_ref, b2_ref, bnp_ref,
          out_ref, y1_all, y2_all, adj_bf, acc, aff, *, nb, count):
    i = pl.program_id(0)
    f32 = jnp.float32

    def stats1_of(b_idx):
        z = y1_all[pl.ds(b_idx, 1)][0].astype(f32)
        y = jnp.maximum(z + b1_ref[...], 0.0)
        return (jnp.sum(y, axis=1, keepdims=True),
                jnp.sum(y * y, axis=1, keepdims=True))

    def stats2_of(b_idx):
        z = y2_all[pl.ds(b_idx, 1)][0].astype(f32)
        y = jnp.maximum(z + b2_ref[...], 0.0)
        return (jnp.sum(y, axis=1, keepdims=True),
                jnp.sum(y * y, axis=1, keepdims=True))

    def produce2(q, stashed):
        z = y1_all[pl.ds(q, 1)][0].astype(f32)
        h = (jnp.maximum(z + b1_ref[...], 0.0) * aff[:, 0:1] + aff[:, 1:2])
        s2v = jnp.dot(h, w2_ref[...], preferred_element_type=f32)
        if stashed:
            raw2 = jnp.dot(adj_bf[pl.ds(q - 1, 1)][0],
                           s2v.astype(jnp.bfloat16),
                           preferred_element_type=f32)
        else:
            raw2 = jnp.dot(adj_ref[0], s2v, preferred_element_type=f32)
        y2_all[pl.ds(q, 1)] = raw2[None].astype(jnp.bfloat16)

    def normalize(q):
        z = y2_all[pl.ds(q, 1)][0].astype(f32)
        out_ref[0] = (jnp.maximum(z + b2_ref[...], 0.0) * aff[:, 2:3]
                      + aff[:, 3:4])

    @pl.when(i < nb)
    def _phase0():
        ps, pq = stats1_of(jnp.maximum(i - 1, 0))
        acc[:, 0:1] = (jnp.where(i == 0, 0.0, acc[:, 0:1])
                       + jnp.where(i >= 1, ps, 0.0))
        acc[:, 1:2] = (jnp.where(i == 0, 0.0, acc[:, 1:2])
                       + jnp.where(i >= 1, pq, 0.0))
        s = jnp.dot(x_ref[0], w1_ref[...], preferred_element_type=f32)
        raw = jnp.dot(adj_ref[0], s, preferred_element_type=f32)
        y1_all[pl.ds(i, 1)] = raw[None].astype(jnp.bfloat16)

        @pl.when((i >= 1) & (i <= nb - 2))
        def _stash():
            adj_bf[pl.ds(i - 1, 1)] = adj_ref[...].astype(jnp.bfloat16)

    @pl.when(i == nb)
    def _fin1():
        ps, pq = stats1_of(nb - 1)
        inv = 1.0 / count
        mean = (acc[:, 0:1] + ps) * inv
        var = (acc[:, 1:2] + pq) * inv - mean * mean
        a = bnp_ref[:, 0:1] * jax.lax.rsqrt(var + EPS)
        aff[:, 0:1] = a
        aff[:, 1:2] = bnp_ref[:, 1:2] - mean * a
        produce2(nb - 1, stashed=False)

    @pl.when(i == nb + 1)
    def _phase1_first():
        ps, pq = stats2_of(nb - 1)
        acc[:, 2:3] = ps
        acc[:, 3:4] = pq
        produce2(0, stashed=False)

    @pl.when((i > nb + 1) & (i < 2 * nb))
    def _phase1():
        j = i - nb
        ps, pq = stats2_of(j - 2)
        acc[:, 2:3] += ps
        acc[:, 3:4] += pq
        produce2(j - 1, stashed=True)

    @pl.when(i == 2 * nb)
    def _fin2():
        ps, pq = stats2_of(nb - 2)
        inv = 1.0 / count
        mean = (acc[:, 2:3] + ps) * inv
        var = (acc[:, 3:4] + pq) * inv - mean * mean
        a = bnp_ref[:, 2:3] * jax.lax.rsqrt(var + EPS)
        aff[:, 2:3] = a
        aff[:, 3:4] = bnp_ref[:, 3:4] - mean * a
        normalize(0)

    @pl.when(i > 2 * nb)
    def _norm():
        normalize(i - 2 * nb)


@jax.jit
def kernel(x, adj, W1, b1, W2, b2, gamma1, beta1, gamma2, beta2):
    B, N, C_in = x.shape
    C_hid = W1.shape[1]
    C_out = W2.shape[1]
    f32 = jnp.float32
    nb = B

    full = lambda shape: pl.BlockSpec(shape, lambda i: (0,) * len(shape))

    def adj_idx(i):
        return (jnp.where(i < nb, i, jnp.where(i == nb, nb - 1, 0)), 0, 0)

    bn_params = jnp.stack([gamma1, beta1, gamma2, beta2], axis=1)

    out = pl.pallas_call(
        functools.partial(_body, nb=nb, count=B * C_hid),
        grid=(3 * nb,),
        in_specs=[
            pl.BlockSpec((1, N, C_in), lambda i: (jnp.minimum(i, nb - 1), 0, 0)),
            pl.BlockSpec((1, N, N), adj_idx),
            full((C_in, C_hid)), full((1, C_hid)),
            full((C_hid, C_out)), full((1, C_out)),
            full((N, 4)),
        ],
        out_specs=pl.BlockSpec(
            (1, N, C_out), lambda i: (jnp.maximum(i - 2 * nb, 0), 0, 0)),
        out_shape=jax.ShapeDtypeStruct((B, N, C_out), f32),
        scratch_shapes=[
            pltpu.VMEM((B, N, C_hid), jnp.bfloat16),
            pltpu.VMEM((B, N, C_out), jnp.bfloat16),
            pltpu.VMEM((B - 2, N, N), jnp.bfloat16),
            pltpu.VMEM((N, 4), f32),
            pltpu.VMEM((N, 4), f32),
        ],
        compiler_params=pltpu.CompilerParams(
            vmem_limit_bytes=100 * 1024 * 1024),
    )(x, adj, W1, b1.reshape(1, C_hid), W2, b2.reshape(1, C_out), bn_params)

    return out


# speedup vs baseline: 1.2252x; 1.0066x over previous
import functools

import jax
import jax.numpy as jnp
from jax.experimental import pallas as pl
from jax.experimental.pallas import tpu as pltpu

EPS = 1e-5


def _body(x_ref, adj_ref, w1_ref, b1_ref, w2_ref, b2_ref, bnp_ref,
          out_ref, y1_all, y2_all, adj_bf, acc, aff, *, nb, count):
    i = pl.program_id(0)
    f32 = jnp.float32

    def stats1_of(b_idx):
        y = y1_all[pl.ds(b_idx, 1)][0].astype(f32)
        return (jnp.sum(y, axis=1, keepdims=True),
                jnp.sum(y * y, axis=1, keepdims=True))

    def stats2_of(b_idx):
        y = y2_all[pl.ds(b_idx, 1)][0].astype(f32)
        return (jnp.sum(y, axis=1, keepdims=True),
                jnp.sum(y * y, axis=1, keepdims=True))

    def produce2(q, stashed):
        y = y1_all[pl.ds(q, 1)][0].astype(f32)
        h = y * aff[:, 0:1] + aff[:, 1:2]
        s2v = jnp.dot(h, w2_ref[...], preferred_element_type=f32)
        if stashed:
            raw2 = jnp.dot(adj_bf[pl.ds(q - 1, 1)][0],
                           s2v.astype(jnp.bfloat16),
                           preferred_element_type=f32)
        else:
            raw2 = jnp.dot(adj_ref[0], s2v, preferred_element_type=f32)
        y2 = jnp.maximum(raw2 + b2_ref[...], 0.0)
        y2_all[pl.ds(q, 1)] = y2[None].astype(jnp.bfloat16)

    def normalize(q):
        y = y2_all[pl.ds(q, 1)][0].astype(f32)
        out_ref[0] = y * aff[:, 2:3] + aff[:, 3:4]

    @pl.when(i < nb)
    def _phase0():
        ps, pq = stats1_of(jnp.maximum(i - 1, 0))
        acc[:, 0:1] = (jnp.where(i == 0, 0.0, acc[:, 0:1])
                       + jnp.where(i >= 1, ps, 0.0))
        acc[:, 1:2] = (jnp.where(i == 0, 0.0, acc[:, 1:2])
                       + jnp.where(i >= 1, pq, 0.0))
        s = jnp.dot(x_ref[0], w1_ref[...], preferred_element_type=f32)
        raw = jnp.dot(adj_ref[0], s, preferred_element_type=f32)
        y1 = jnp.maximum(raw + b1_ref[...], 0.0)
        y1_all[pl.ds(i, 1)] = y1[None].astype(jnp.bfloat16)

        @pl.when((i >= 1) & (i <= nb - 2))
        def _stash():
            adj_bf[pl.ds(i - 1, 1)] = adj_ref[...].astype(jnp.bfloat16)

    @pl.when(i == nb)
    def _fin1():
        ps, pq = stats1_of(nb - 1)
        inv = 1.0 / count
        mean = (acc[:, 0:1] + ps) * inv
        var = (acc[:, 1:2] + pq) * inv - mean * mean
        a = bnp_ref[:, 0:1] * jax.lax.rsqrt(var + EPS)
        aff[:, 0:1] = a
        aff[:, 1:2] = bnp_ref[:, 1:2] - mean * a
        produce2(nb - 1, stashed=False)

    @pl.when(i == nb + 1)
    def _phase1_first():
        ps, pq = stats2_of(nb - 1)
        acc[:, 2:3] = ps
        acc[:, 3:4] = pq
        produce2(0, stashed=False)

    @pl.when((i > nb + 1) & (i < 2 * nb))
    def _phase1():
        j = i - nb
        ps, pq = stats2_of(j - 2)
        acc[:, 2:3] += ps
        acc[:, 3:4] += pq
        produce2(j - 1, stashed=True)

    @pl.when(i == 2 * nb)
    def _fin2():
        ps, pq = stats2_of(nb - 2)
        inv = 1.0 / count
        mean = (acc[:, 2:3] + ps) * inv
        var = (acc[:, 3:4] + pq) * inv - mean * mean
        a = bnp_ref[:, 2:3] * jax.lax.rsqrt(var + EPS)
        aff[:, 2:3] = a
        aff[:, 3:4] = bnp_ref[:, 3:4] - mean * a
        normalize(0)

    @pl.when(i > 2 * nb)
    def _norm():
        normalize(i - 2 * nb)


@jax.jit
def kernel(x, adj, W1, b1, W2, b2, gamma1, beta1, gamma2, beta2):
    B, N, C_in = x.shape
    C_hid = W1.shape[1]
    C_out = W2.shape[1]
    f32 = jnp.float32
    nb = B

    full = lambda shape: pl.BlockSpec(shape, lambda i: (0,) * len(shape))

    def adj_idx(i):
        return (jnp.where(i < nb, i, jnp.where(i == nb, nb - 1, 0)), 0, 0)

    bn_params = jnp.stack([gamma1, beta1, gamma2, beta2], axis=1)

    out = pl.pallas_call(
        functools.partial(_body, nb=nb, count=B * C_hid),
        grid=(3 * nb,),
        in_specs=[
            pl.BlockSpec((1, N, C_in), lambda i: (jnp.minimum(i, nb - 1), 0, 0)),
            pl.BlockSpec((1, N, N), adj_idx),
            full((C_in, C_hid)), full((1, C_hid)),
            full((C_hid, C_out)), full((1, C_out)),
            full((N, 4)),
        ],
        out_specs=pl.BlockSpec(
            (1, N, C_out), lambda i: (jnp.maximum(i - 2 * nb, 0), 0, 0)),
        out_shape=jax.ShapeDtypeStruct((B, N, C_out), f32),
        scratch_shapes=[
            pltpu.VMEM((B, N, C_hid), jnp.bfloat16),
            pltpu.VMEM((B, N, C_out), jnp.bfloat16),
            pltpu.VMEM((B - 2, N, N), jnp.bfloat16),
            pltpu.VMEM((N, 4), f32),
            pltpu.VMEM((N, 4), f32),
        ],
        compiler_params=pltpu.CompilerParams(
            vmem_limit_bytes=100 * 1024 * 1024),
    )(x, adj, W1, b1.reshape(1, C_hid), W2, b2.reshape(1, C_out), bn_params)

    return out


# speedup vs baseline: 1.2726x; 1.0387x over previous
import functools

import jax
import jax.numpy as jnp
from jax.experimental import pallas as pl
from jax.experimental.pallas import tpu as pltpu

EPS = 1e-5


def _body(x_ref, adj_ref, w1_ref, b1_ref, w2_ref, b2_ref, bnp_ref,
          out_ref, y1_all, y2_all, adj_bf, acc, aff, *, nb, count):
    i = pl.program_id(0)
    f32 = jnp.float32

    def stats1_of(b_idx):
        y = y1_all[pl.ds(b_idx, 1)][0].astype(f32)
        return (jnp.sum(y, axis=1, keepdims=True),
                jnp.sum(y * y, axis=1, keepdims=True))

    def stats2_of(b_idx):
        y = y2_all[pl.ds(b_idx, 1)][0].astype(f32)
        return (jnp.sum(y, axis=1, keepdims=True),
                jnp.sum(y * y, axis=1, keepdims=True))

    def produce2(q, stashed):
        y = y1_all[pl.ds(q, 1)][0].astype(f32)
        h = y * aff[:, 0:1] + aff[:, 1:2]
        s2v = jnp.dot(h, w2_ref[...], preferred_element_type=f32)
        if stashed:
            raw2 = jnp.dot(adj_bf[pl.ds(q - 1, 1)][0],
                           s2v.astype(jnp.bfloat16),
                           preferred_element_type=f32)
        else:
            raw2 = jnp.dot(adj_ref[0], s2v, preferred_element_type=f32)
        y2 = jnp.maximum(raw2 + b2_ref[...], 0.0)
        y2_all[pl.ds(q, 1)] = y2[None].astype(jnp.bfloat16)

    def normalize2(base):
        for k in range(2):
            y = y2_all[pl.ds(base + k, 1)][0].astype(f32)
            out_ref[k] = y * aff[:, 2:3] + aff[:, 3:4]

    @pl.when(i < nb)
    def _phase0():
        ps, pq = stats1_of(jnp.maximum(i - 1, 0))
        acc[:, 0:1] = (jnp.where(i == 0, 0.0, acc[:, 0:1])
                       + jnp.where(i >= 1, ps, 0.0))
        acc[:, 1:2] = (jnp.where(i == 0, 0.0, acc[:, 1:2])
                       + jnp.where(i >= 1, pq, 0.0))
        s = jnp.dot(x_ref[0], w1_ref[...], preferred_element_type=f32)
        raw = jnp.dot(adj_ref[0], s, preferred_element_type=f32)
        y1 = jnp.maximum(raw + b1_ref[...], 0.0)
        y1_all[pl.ds(i, 1)] = y1[None].astype(jnp.bfloat16)

        @pl.when((i >= 1) & (i <= nb - 2))
        def _stash():
            adj_bf[pl.ds(i - 1, 1)] = adj_ref[...].astype(jnp.bfloat16)

    @pl.when(i == nb)
    def _fin1():
        ps, pq = stats1_of(nb - 1)
        inv = 1.0 / count
        mean = (acc[:, 0:1] + ps) * inv
        var = (acc[:, 1:2] + pq) * inv - mean * mean
        a = bnp_ref[:, 0:1] * jax.lax.rsqrt(var + EPS)
        aff[:, 0:1] = a
        aff[:, 1:2] = bnp_ref[:, 1:2] - mean * a
        produce2(nb - 1, stashed=False)

    @pl.when(i == nb + 1)
    def _phase1_first():
        ps, pq = stats2_of(nb - 1)
        acc[:, 2:3] = ps
        acc[:, 3:4] = pq
        produce2(0, stashed=False)

    @pl.when((i > nb + 1) & (i < 2 * nb))
    def _phase1():
        j = i - nb
        ps, pq = stats2_of(j - 2)
        acc[:, 2:3] += ps
        acc[:, 3:4] += pq
        produce2(j - 1, stashed=True)

    @pl.when(i == 2 * nb)
    def _fin2():
        ps, pq = stats2_of(nb - 2)
        inv = 1.0 / count
        mean = (acc[:, 2:3] + ps) * inv
        var = (acc[:, 3:4] + pq) * inv - mean * mean
        a = bnp_ref[:, 2:3] * jax.lax.rsqrt(var + EPS)
        aff[:, 2:3] = a
        aff[:, 3:4] = bnp_ref[:, 3:4] - mean * a
        normalize2(0)

    @pl.when(i > 2 * nb)
    def _norm():
        normalize2(2 * (i - 2 * nb))


@jax.jit
def kernel(x, adj, W1, b1, W2, b2, gamma1, beta1, gamma2, beta2):
    B, N, C_in = x.shape
    C_hid = W1.shape[1]
    C_out = W2.shape[1]
    f32 = jnp.float32
    nb = B

    full = lambda shape: pl.BlockSpec(shape, lambda i: (0,) * len(shape))

    def adj_idx(i):
        return (jnp.where(i < nb, i, jnp.where(i == nb, nb - 1, 0)), 0, 0)

    bn_params = jnp.stack([gamma1, beta1, gamma2, beta2], axis=1)

    out = pl.pallas_call(
        functools.partial(_body, nb=nb, count=B * C_hid),
        grid=(2 * nb + nb // 2,),
        in_specs=[
            pl.BlockSpec((1, N, C_in), lambda i: (jnp.minimum(i, nb - 1), 0, 0)),
            pl.BlockSpec((1, N, N), adj_idx),
            full((C_in, C_hid)), full((1, C_hid)),
            full((C_hid, C_out)), full((1, C_out)),
            full((N, 4)),
        ],
        out_specs=pl.BlockSpec(
            (2, N, C_out), lambda i: (jnp.maximum(i - 2 * nb, 0), 0, 0)),
        out_shape=jax.ShapeDtypeStruct((B, N, C_out), f32),
        scratch_shapes=[
            pltpu.VMEM((B, N, C_hid), jnp.bfloat16),
            pltpu.VMEM((B, N, C_out), jnp.bfloat16),
            pltpu.VMEM((B - 2, N, N), jnp.bfloat16),
            pltpu.VMEM((N, 4), f32),
            pltpu.VMEM((N, 4), f32),
        ],
        compiler_params=pltpu.CompilerParams(
            vmem_limit_bytes=100 * 1024 * 1024),
    )(x, adj, W1, b1.reshape(1, C_hid), W2, b2.reshape(1, C_out), bn_params)

    return out
